# baseline (speedup 1.0000x reference)
import functools

import jax
import jax.numpy as jnp
from jax import lax
from jax.experimental import pallas as pl
from jax.experimental.pallas import tpu as pltpu
from jax.experimental.pallas import tpu_sc as plsc

N_NODES = 10000
D = 128
N_EDGES = 320000

NC = 2
NS = 16
NW = NC * NS
EPW = N_EDGES // NW
C = 80
NCH = EPW // C
GPC = C // 16
NBUF = 4
DW = D // 2
DU = 8


def _sc_body(z_h, u_h, v_h, out_h, idxu, idxv,
             zu0, zu1, zu2, zu3, zv0, zv1, zv2, zv3, outf, pbuf,
             su0, su1, su2, su3, sv0, sv1, sv2, sv3):
    zus = (zu0, zu1, zu2, zu3)
    zvs = (zv0, zv1, zv2, zv3)
    sus = (su0, su1, su2, su3)
    svs = (sv0, sv1, sv2, sv3)

    cid = lax.axis_index("c")
    sid = lax.axis_index("s")
    wid = sid * NC + cid

    pltpu.sync_copy(u_h.at[wid], idxu)
    pltpu.sync_copy(v_h.at[wid], idxv)

    lanes = lax.iota(jnp.int32, 16)

    def issue(k, b):
        pltpu.async_copy(z_h.at[idxu.at[k]], zus[b], sus[b])
        pltpu.async_copy(z_h.at[idxv.at[k]], zvs[b], svs[b])

    def wait(k, b):
        pltpu.make_async_copy(z_h.at[idxu.at[k]], zus[b], sus[b]).wait()
        pltpu.make_async_copy(z_h.at[idxv.at[k]], zvs[b], svs[b]).wait()

    def compute(k, b):
        zu = zus[b]
        zv = zvs[b]

        def gbody(g, carry):
            eids = g * 16 + lanes

            def dbody(dd, accs):
                news = list(accs)
                for d2 in range(DU):
                    colv = lax.broadcast(dd * DU + d2, (16,))
                    pu = plsc.load_gather(zu, [eids, colv])
                    pv = plsc.load_gather(zv, [eids, colv])
                    au0, au1 = plsc.unpack(
                        plsc.bitcast(pu, jnp.bfloat16),
                        format=plsc.PackFormat.INTERLEAVED)
                    av0, av1 = plsc.unpack(
                        plsc.bitcast(pv, jnp.bfloat16),
                        format=plsc.PackFormat.INTERLEAVED)
                    news[(2 * d2) % 4] = news[(2 * d2) % 4] + au0 * av0
                    news[(2 * d2 + 1) % 4] = news[(2 * d2 + 1) % 4] + au1 * av1
                return tuple(news)

            zv16 = jnp.zeros((16,), jnp.float32)
            accs = lax.fori_loop(0, DW // DU, dbody, (zv16, zv16, zv16, zv16))
            acc = (accs[0] + accs[1]) + (accs[2] + accs[3])
            plsc.store_scatter(outf, [k * C + g * 16 + lanes], acc)
            return carry

        lax.fori_loop(0, GPC, gbody, 0)

    def probe_body(i, carry):
        pltpu.sync_copy(z_h.at[pl.ds((i % 16) * 625, 625)], pbuf)
        return carry
    lax.fori_loop(0, 80, probe_body, 0)

    for b in range(NBUF):
        issue(b, b)

    def quadbody(j, carry):
        for b in range(NBUF):
            k = NBUF * j + b
            wait(k, b)
            compute(k, b)
            nk = k + NBUF

            @pl.when(nk < NCH)
            def _():
                issue(nk, b)
        return carry

    lax.fori_loop(0, NCH // NBUF, quadbody, 0)

    for k in range(NBUF * (NCH // NBUF), NCH):
        b = k % NBUF
        wait(k, b)
        compute(k, b)

    pltpu.sync_copy(outf, out_h.at[pl.ds(wid * EPW, EPW)])


@jax.jit
def _decode(z, u3, v3):
    mesh = plsc.VectorSubcoreMesh(core_axis_name="c", subcore_axis_name="s")
    return pl.kernel(
        _sc_body,
        mesh=mesh,
        compiler_params=pltpu.CompilerParams(needs_layout_passes=False, use_tc_tiling_on_sc=False),
        out_type=jax.ShapeDtypeStruct((N_EDGES,), jnp.float32),
        scratch_types=[
            pltpu.VMEM((NCH, C), jnp.int32),
            pltpu.VMEM((NCH, C), jnp.int32),
        ] + [pltpu.VMEM((C, DW), jnp.int32)] * (2 * NBUF) + [
            pltpu.VMEM((EPW,), jnp.float32),
            pltpu.VMEM((625, DW), jnp.int32),
        ] + [pltpu.SemaphoreType.DMA] * (2 * NBUF),
    )(z, u3, v3)


def kernel(z, edge_index_query):
    eiq = edge_index_query.astype(jnp.int32)
    u3 = eiq[0].reshape(NW, NCH, C)
    v3 = eiq[1].reshape(NW, NCH, C)
    z_pk = lax.bitcast_convert_type(
        z.astype(jnp.bfloat16).reshape(N_NODES, DW, 2), jnp.int32)
    return _decode(z_pk, u3, v3)


# speedup vs baseline: 1.3591x; 1.3591x over previous
import functools

import jax
import jax.numpy as jnp
from jax import lax
from jax.experimental import pallas as pl
from jax.experimental.pallas import tpu as pltpu
from jax.experimental.pallas import tpu_sc as plsc

N_NODES = 10000
D = 128
N_EDGES = 320000

NC = 2
NS = 16
NW = NC * NS
EPW = N_EDGES // NW
C = 80
NCH = EPW // C
GPC = C // 16
NBUF = 4
DW = D // 2
DU = 8


def _sc_body(z_h, u_h, v_h, out_h, idxu, idxv,
             zu0, zu1, zu2, zu3, zv0, zv1, zv2, zv3, outf,
             su0, su1, su2, su3, sv0, sv1, sv2, sv3):
    zus = (zu0, zu1, zu2, zu3)
    zvs = (zv0, zv1, zv2, zv3)
    sus = (su0, su1, su2, su3)
    svs = (sv0, sv1, sv2, sv3)

    cid = lax.axis_index("c")
    sid = lax.axis_index("s")
    wid = sid * NC + cid

    pltpu.sync_copy(u_h.at[wid], idxu)
    pltpu.sync_copy(v_h.at[wid], idxv)

    lanes = lax.iota(jnp.int32, 16)

    def issue(k, b):
        pltpu.async_copy(z_h.at[idxu.at[k]], zus[b], sus[b])
        pltpu.async_copy(z_h.at[idxv.at[k]], zvs[b], svs[b])

    def wait(k, b):
        pltpu.make_async_copy(z_h.at[idxu.at[k]], zus[b], sus[b]).wait()
        pltpu.make_async_copy(z_h.at[idxv.at[k]], zvs[b], svs[b]).wait()

    def compute(k, b):
        zu = zus[b]
        zv = zvs[b]

        def gbody(g, carry):
            eids = g * 16 + lanes

            def dbody(dd, accs):
                news = list(accs)
                for d2 in range(DU):
                    colv = lax.broadcast(dd * DU + d2, (16,))
                    pu = plsc.load_gather(zu, [eids, colv])
                    pv = plsc.load_gather(zv, [eids, colv])
                    au0, au1 = plsc.unpack(
                        plsc.bitcast(pu, jnp.bfloat16),
                        format=plsc.PackFormat.INTERLEAVED)
                    av0, av1 = plsc.unpack(
                        plsc.bitcast(pv, jnp.bfloat16),
                        format=plsc.PackFormat.INTERLEAVED)
                    news[(2 * d2) % 4] = news[(2 * d2) % 4] + au0 * av0
                    news[(2 * d2 + 1) % 4] = news[(2 * d2 + 1) % 4] + au1 * av1
                return tuple(news)

            zv16 = jnp.zeros((16,), jnp.float32)
            accs = lax.fori_loop(0, DW // DU, dbody, (zv16, zv16, zv16, zv16))
            acc = (accs[0] + accs[1]) + (accs[2] + accs[3])
            plsc.store_scatter(outf, [k * C + g * 16 + lanes], acc)
            return carry

        lax.fori_loop(0, GPC, gbody, 0)

    for b in range(NBUF):
        issue(b, b)

    def quadbody(j, carry):
        for b in range(NBUF):
            k = NBUF * j + b
            wait(k, b)
            compute(k, b)
            nk = k + NBUF

            @pl.when(nk < NCH)
            def _():
                issue(nk, b)
        return carry

    lax.fori_loop(0, NCH // NBUF, quadbody, 0)

    for k in range(NBUF * (NCH // NBUF), NCH):
        b = k % NBUF
        wait(k, b)
        compute(k, b)

    pltpu.sync_copy(outf, out_h.at[pl.ds(wid * EPW, EPW)])


@jax.jit
def _decode(z, u3, v3):
    mesh = plsc.VectorSubcoreMesh(core_axis_name="c", subcore_axis_name="s")
    return pl.kernel(
        _sc_body,
        mesh=mesh,
        compiler_params=pltpu.CompilerParams(needs_layout_passes=False, use_tc_tiling_on_sc=False),
        out_type=jax.ShapeDtypeStruct((N_EDGES,), jnp.float32),
        scratch_types=[
            pltpu.VMEM((NCH, C), jnp.int32),
            pltpu.VMEM((NCH, C), jnp.int32),
        ] + [pltpu.VMEM((C, DW), jnp.int32)] * (2 * NBUF) + [
            pltpu.VMEM((EPW,), jnp.float32),
        ] + [pltpu.SemaphoreType.DMA] * (2 * NBUF),
    )(z, u3, v3)


def kernel(z, edge_index_query):
    eiq = edge_index_query.astype(jnp.int32)
    u3 = eiq[0].reshape(NW, NCH, C)
    v3 = eiq[1].reshape(NW, NCH, C)
    z_pk = lax.bitcast_convert_type(
        z.astype(jnp.bfloat16).reshape(N_NODES, DW, 2), jnp.int32)
    return _decode(z_pk, u3, v3)


# speedup vs baseline: 4.3428x; 3.1955x over previous
import functools

import jax
import jax.numpy as jnp
from jax import lax
from jax.experimental import pallas as pl
from jax.experimental.pallas import tpu as pltpu
from jax.experimental.pallas import tpu_sc as plsc

N_NODES = 10000
D = 128
N_EDGES = 320000

NC = 2
NS = 16
NW = NC * NS
EPW = N_EDGES // NW
C = 80
NCH = EPW // C
GPC = C // 16
NBUF = 4
DW = D // 2
DWP = 72
DU = 8


def _sc_body(z_h, u_h, v_h, out_h, idxu, idxv,
             zu0, zu1, zu2, zu3, zv0, zv1, zv2, zv3, outf,
             su0, su1, su2, su3, sv0, sv1, sv2, sv3):
    zus = (zu0, zu1, zu2, zu3)
    zvs = (zv0, zv1, zv2, zv3)
    sus = (su0, su1, su2, su3)
    svs = (sv0, sv1, sv2, sv3)

    cid = lax.axis_index("c")
    sid = lax.axis_index("s")
    wid = sid * NC + cid

    pltpu.sync_copy(u_h.at[wid], idxu)
    pltpu.sync_copy(v_h.at[wid], idxv)

    lanes = lax.iota(jnp.int32, 16)

    def issue(k, b):
        pltpu.async_copy(z_h.at[idxu.at[k]], zus[b], sus[b])
        pltpu.async_copy(z_h.at[idxv.at[k]], zvs[b], svs[b])

    def wait(k, b):
        pltpu.make_async_copy(z_h.at[idxu.at[k]], zus[b], sus[b]).wait()
        pltpu.make_async_copy(z_h.at[idxv.at[k]], zvs[b], svs[b]).wait()

    def compute(k, b):
        zu = zus[b]
        zv = zvs[b]

        def gbody(g, carry):
            eids = g * 16 + lanes

            def dbody(dd, accs):
                news = list(accs)
                for d2 in range(DU):
                    colv = lax.broadcast(dd * DU + d2, (16,))
                    pu = plsc.load_gather(zu, [eids, colv])
                    pv = plsc.load_gather(zv, [eids, colv])
                    au0, au1 = plsc.unpack(
                        plsc.bitcast(pu, jnp.bfloat16),
                        format=plsc.PackFormat.INTERLEAVED)
                    av0, av1 = plsc.unpack(
                        plsc.bitcast(pv, jnp.bfloat16),
                        format=plsc.PackFormat.INTERLEAVED)
                    news[(2 * d2) % 4] = news[(2 * d2) % 4] + au0 * av0
                    news[(2 * d2 + 1) % 4] = news[(2 * d2 + 1) % 4] + au1 * av1
                return tuple(news)

            zv16 = jnp.zeros((16,), jnp.float32)
            accs = lax.fori_loop(0, DW // DU, dbody, (zv16, zv16, zv16, zv16))
            acc = (accs[0] + accs[1]) + (accs[2] + accs[3])
            plsc.store_scatter(outf, [k * C + g * 16 + lanes], acc)
            return carry

        lax.fori_loop(0, GPC, gbody, 0)

    for b in range(NBUF):
        issue(b, b)

    def quadbody(j, carry):
        for b in range(NBUF):
            k = NBUF * j + b
            wait(k, b)
            compute(k, b)
            nk = k + NBUF

            @pl.when(nk < NCH)
            def _():
                issue(nk, b)
        return carry

    lax.fori_loop(0, NCH // NBUF, quadbody, 0)

    for k in range(NBUF * (NCH // NBUF), NCH):
        b = k % NBUF
        wait(k, b)
        compute(k, b)

    pltpu.sync_copy(outf, out_h.at[pl.ds(wid * EPW, EPW)])


@jax.jit
def _decode(z, u3, v3):
    mesh = plsc.VectorSubcoreMesh(core_axis_name="c", subcore_axis_name="s")
    return pl.kernel(
        _sc_body,
        mesh=mesh,
        compiler_params=pltpu.CompilerParams(needs_layout_passes=False, use_tc_tiling_on_sc=False),
        out_type=jax.ShapeDtypeStruct((N_EDGES,), jnp.float32),
        scratch_types=[
            pltpu.VMEM((NCH, C), jnp.int32),
            pltpu.VMEM((NCH, C), jnp.int32),
        ] + [pltpu.VMEM((C, DWP), jnp.int32)] * (2 * NBUF) + [
            pltpu.VMEM((EPW,), jnp.float32),
        ] + [pltpu.SemaphoreType.DMA] * (2 * NBUF),
    )(z, u3, v3)


def kernel(z, edge_index_query):
    eiq = edge_index_query.astype(jnp.int32)
    u3 = eiq[0].reshape(NW, NCH, C)
    v3 = eiq[1].reshape(NW, NCH, C)
    z_pk = lax.bitcast_convert_type(
        z.astype(jnp.bfloat16).reshape(N_NODES, DW, 2), jnp.int32)
    z_pk = jnp.pad(z_pk, ((0, 0), (0, DWP - DW)))
    return _decode(z_pk, u3, v3)
